# Initial kernel scaffold; baseline (speedup 1.0000x reference)
import jax
import jax.numpy as jnp
from jax.experimental import pallas as pl


def kernel(items, inputs, alias_inputs, edge_index, edge_weight, item_table, Wq, bq, Wk, bk, Wv, bv):
    raise NotImplementedError("write your pallas kernel here")


# speedup vs baseline: 1.0027x; 1.0027x over previous
import functools

import jax
import jax.numpy as jnp
from jax.experimental import pallas as pl

DIM = 128
NUM_NODE = 10000
N = NUM_NODE + 1
LAYERS = 2
W_K = 12.0
B = 1024
L = 50
E = 320000

N_PAD = 10240


def _attn_body(seq_ref, wq_ref, bq_ref, wk_ref, bk_ref, wv_ref, bv_ref, out_ref):
    seq = seq_ref[...]
    sess = jnp.sum(seq, axis=1)
    q = jnp.dot(sess, wq_ref[...], preferred_element_type=jnp.float32) + bq_ref[...]
    bb = seq.shape[0]
    seq2 = seq.reshape(bb * L, DIM)
    k = jnp.dot(seq2, wk_ref[...], preferred_element_type=jnp.float32) + bk_ref[...]
    v = jnp.dot(seq2, wv_ref[...], preferred_element_type=jnp.float32) + bv_ref[...]
    k3 = k.reshape(bb, L, DIM)
    v3 = v.reshape(bb, L, DIM)
    scale = DIM ** 0.5
    att = jnp.sum(q[:, None, :] * k3, axis=-1) / scale
    att = jax.nn.softmax(att, axis=-1)
    sess_emb = jnp.sum(att[:, :, None] * v3, axis=1)
    nrm = jnp.sqrt(jnp.sum(sess_emb * sess_emb, axis=-1, keepdims=True))
    out_ref[...] = W_K * sess_emb / jnp.maximum(nrm, 1e-12)


def _attention_select(seq_hidden, Wq, bq, Wk, bk, Wv, bv):
    bblk = 128
    grid = (B // bblk,)
    return pl.pallas_call(
        _attn_body,
        grid=grid,
        in_specs=[
            pl.BlockSpec((bblk, L, DIM), lambda i: (i, 0, 0)),
            pl.BlockSpec((DIM, DIM), lambda i: (0, 0)),
            pl.BlockSpec((1, DIM), lambda i: (0, 0)),
            pl.BlockSpec((DIM, DIM), lambda i: (0, 0)),
            pl.BlockSpec((1, DIM), lambda i: (0, 0)),
            pl.BlockSpec((DIM, DIM), lambda i: (0, 0)),
            pl.BlockSpec((1, DIM), lambda i: (0, 0)),
        ],
        out_specs=pl.BlockSpec((bblk, DIM), lambda i: (i, 0)),
        out_shape=jax.ShapeDtypeStruct((B, DIM), jnp.float32),
    )(seq_hidden, Wq, bq.reshape(1, DIM), Wk, bk.reshape(1, DIM),
      Wv, bv.reshape(1, DIM))


def _scores_body(sel_ref, emb_ref, out_ref):
    emb = emb_ref[...]
    nrm = jnp.sqrt(jnp.sum(emb * emb, axis=-1, keepdims=True))
    norm = emb / jnp.maximum(nrm, 1e-12)
    out_ref[...] = jax.lax.dot_general(
        sel_ref[...], norm,
        dimension_numbers=(((1,), (1,)), ((), ())),
        preferred_element_type=jnp.float32)


def _scores(select, emb_pad):
    bblk, nblk = 256, 1280
    grid = (B // bblk, N_PAD // nblk)
    return pl.pallas_call(
        _scores_body,
        grid=grid,
        in_specs=[
            pl.BlockSpec((bblk, DIM), lambda i, j: (i, 0)),
            pl.BlockSpec((nblk, DIM), lambda i, j: (j, 0)),
        ],
        out_specs=pl.BlockSpec((bblk, nblk), lambda i, j: (i, j)),
        out_shape=jax.ShapeDtypeStruct((B, N_PAD), jnp.float32),
    )(select, emb_pad)


def kernel(items, inputs, alias_inputs, edge_index, edge_weight, item_table,
           Wq, bq, Wk, bk, Wv, bv):
    h = item_table
    acc = item_table
    for _ in range(LAYERS):
        msgs = h[edge_index[1]] * edge_weight[:, None]
        h = jax.ops.segment_sum(msgs, edge_index[0], num_segments=N)
        acc = acc + h
    emb = acc / (LAYERS + 1)

    combined = jnp.take_along_axis(items, alias_inputs, axis=1)
    seq_hidden = emb[combined]

    select = _attention_select(seq_hidden, Wq, bq, Wk, bk, Wv, bv)
    emb_pad = jnp.zeros((N_PAD, DIM), jnp.float32).at[:N].set(emb)
    scores = _scores(select, emb_pad)
    return scores[:, :N]


# speedup vs baseline: 3.0107x; 3.0027x over previous
import functools

import jax
import jax.numpy as jnp
from jax import lax
from jax.experimental import pallas as pl
from jax.experimental.pallas import tpu as pltpu
from jax.experimental.pallas import tpu_sc as plsc

DIM = 128
DH = DIM // 2
NUM_NODE = 10000
N = NUM_NODE + 1
LAYERS = 2
W_K = 12.0
B = 1024
L = 50
E = 320000

N_PAD = 10240
E_PAD = 327680
C = 128
NCH = 160
G = 40
NG = NCH // G
NROW = N_PAD // 16
SESS_PT = B // 16


def _sc_body(x_st, srcm, dstm, wm, items_p, alias_p,
             emb_st, h1_st, seq_st,
             src_v, dst_v, w_v, rows0, rows1, rows2,
             items_v, alias_v, idx_v, srow0, srow1,
             acc,
             gsem0, gsem1):
    c = lax.axis_index("c")
    s = lax.axis_index("s")
    z16 = jnp.zeros((16,), jnp.float32)

    def _zero_row(i, _):
        for k in range(DH // 16):
            rows0[i, pl.ds(k * 16, 16)] = z16
        return 0
    lax.fori_loop(0, C, _zero_row, 0)

    def _zero_acc():
        for k in range(NROW // C):
            pltpu.sync_copy(rows0, acc.at[pl.ds(s * NROW + k * C, C)])
    _zero_acc()

    chunk0 = s * NCH
    coff = c * N_PAD

    plsc.subcore_barrier()

    def _gather_start(table, j, buf, sem):
        return pltpu.async_copy(table.at[src_v.at[j]], buf, sem)

    def _gather_wait(table, buf, sem):
        pltpu.make_async_copy(table.at[pl.ds(0, C)], buf, sem).wait()

    def _scale(buf, lj):
        def _grp(g, _):
            for e16 in range(16):
                ei = g * 16 + e16
                wv = plsc.load_gather(
                    w_v, [jnp.full((16,), lj * C + ei, jnp.int32)])
                for k in range(DH // 16):
                    sl = pl.ds(k * 16, 16)
                    buf[ei, sl] = buf[ei, sl] * wv
            return 0
        lax.fori_loop(0, C // 16, _grp, 0)

    def _layer(table):
        for grp in range(NG):
            g0 = chunk0 + grp * G
            pltpu.sync_copy(srcm.at[pl.ds(g0, G)], src_v)
            pltpu.sync_copy(dstm.at[pl.ds(g0, G)], dst_v)
            pltpu.sync_copy(wm.at[pl.ds(g0 * C, G * C)], w_v)

            def _offs_row(i, _):
                for k in range(C // 16):
                    sl = pl.ds(k * 16, 16)
                    src_v[i, sl] = src_v[i, sl] + coff
                return 0
            lax.fori_loop(0, G, _offs_row, 0)

            _gather_start(table, 0, rows0, gsem0)

            def _pair(i, _):
                j0 = 2 * i
                j1 = 2 * i + 1
                _gather_wait(table, rows0, gsem0)
                _gather_start(table, j1, rows1, gsem1)
                _scale(rows0, j0)
                pltpu.sync_copy(rows0, acc.at[dst_v.at[j0]], add=True)
                _gather_wait(table, rows1, gsem1)

                @pl.when(i < G // 2 - 1)
                def _():
                    _gather_start(table, j1 + 1, rows0, gsem0)
                _scale(rows1, j1)
                pltpu.sync_copy(rows1, acc.at[dst_v.at[j1]], add=True)
                return 0
            lax.fori_loop(0, G // 2, _pair, 0)

    _layer(x_st)
    plsc.subcore_barrier()

    for k in range(NROW // C):
        r0 = s * NROW + k * C
        pltpu.sync_copy(acc.at[pl.ds(r0, C)], rows1)
        pltpu.sync_copy(rows1, h1_st.at[pl.ds(coff + r0, C)])
    lax.fori_loop(0, C, _zero_row, 0)
    _zero_acc()
    plsc.subcore_barrier()

    _layer(h1_st)
    plsc.subcore_barrier()

    third = jnp.float32(1.0 / 3.0)
    for k in range(NROW // C):
        r0 = s * NROW + k * C
        pltpu.sync_copy(x_st.at[pl.ds(coff + r0, C)], rows0)
        pltpu.sync_copy(h1_st.at[pl.ds(coff + r0, C)], rows1)
        pltpu.sync_copy(acc.at[pl.ds(r0, C)], rows2)

        def _comb(i, _):
            for kk in range(DH // 16):
                sl = pl.ds(kk * 16, 16)
                rows0[i, sl] = (rows0[i, sl] + rows1[i, sl]
                                + rows2[i, sl]) * third
            return 0
        lax.fori_loop(0, C, _comb, 0)
        pltpu.sync_copy(rows0, emb_st.at[pl.ds(coff + r0, C)])
    plsc.subcore_barrier()

    sess0 = s * SESS_PT
    pltpu.sync_copy(items_p.at[pl.ds(sess0 * 64, SESS_PT * 64)], items_v)
    pltpu.sync_copy(alias_p.at[pl.ds(sess0, SESS_PT)], alias_v)

    def _mkidx(i, _):
        for g in range(4):
            sl = pl.ds(g * 16, 16)
            av = alias_v[i, sl]
            cb = plsc.load_gather(items_v, [av + i * 64])
            idx_v[i, sl] = cb + coff
        return 0
    lax.fori_loop(0, SESS_PT, _mkidx, 0)

    soff = c * (B * 64)

    def _sgather_start(i, buf, sem):
        return pltpu.async_copy(emb_st.at[idx_v.at[i]], buf, sem)

    def _sgather_wait(buf, sem):
        pltpu.make_async_copy(emb_st.at[pl.ds(0, 64)], buf, sem).wait()

    _sgather_start(0, srow0, gsem0)

    def _sess_pair(i, _):
        i0 = 2 * i
        i1 = 2 * i + 1
        _sgather_wait(srow0, gsem0)
        _sgather_start(i1, srow1, gsem1)
        pltpu.sync_copy(srow0,
                        seq_st.at[pl.ds(soff + (sess0 + i0) * 64, 64)])
        _sgather_wait(srow1, gsem1)

        @pl.when(i < SESS_PT // 2 - 1)
        def _():
            _sgather_start(i1 + 1, srow0, gsem0)
        pltpu.sync_copy(srow1,
                        seq_st.at[pl.ds(soff + (sess0 + i1) * 64, 64)])
        return 0
    lax.fori_loop(0, SESS_PT // 2, _sess_pair, 0)


@jax.jit
def _graph_conv_sc(x_st, srcm, dstm, wm, items_p, alias_p):
    mesh = plsc.VectorSubcoreMesh(core_axis_name="c", subcore_axis_name="s")
    f = pl.kernel(
        _sc_body,
        out_type=(
            jax.ShapeDtypeStruct((2 * N_PAD, DH), jnp.float32),
            jax.ShapeDtypeStruct((2 * N_PAD, DH), jnp.float32),
            jax.ShapeDtypeStruct((2 * B * 64, DH), jnp.float32),
        ),
        mesh=mesh,
        compiler_params=pltpu.CompilerParams(
            needs_layout_passes=False, use_tc_tiling_on_sc=False),
        scratch_types=[
            pltpu.VMEM((G, C), jnp.int32),
            pltpu.VMEM((G, C), jnp.int32),
            pltpu.VMEM((G * C,), jnp.float32),
            pltpu.VMEM((C, DH), jnp.float32),
            pltpu.VMEM((C, DH), jnp.float32),
            pltpu.VMEM((C, DH), jnp.float32),
            pltpu.VMEM((SESS_PT * 64,), jnp.int32),
            pltpu.VMEM((SESS_PT, 64), jnp.int32),
            pltpu.VMEM((SESS_PT, 64), jnp.int32),
            pltpu.VMEM((64, DH), jnp.float32),
            pltpu.VMEM((64, DH), jnp.float32),
            pltpu.VMEM_SHARED((N_PAD, DH), jnp.float32),
            pltpu.SemaphoreType.DMA,
            pltpu.SemaphoreType.DMA,
        ],
    )
    return f(x_st, srcm, dstm, wm, items_p, alias_p)


def _attn_body(seq_ref, wq_ref, bq_ref, wk_ref, bk_ref, wv_ref, bv_ref, out_ref):
    seq = seq_ref[...]
    sess = jnp.sum(seq, axis=1)
    q = jnp.dot(sess, wq_ref[...], preferred_element_type=jnp.float32) + bq_ref[...]
    bb = seq.shape[0]
    seq2 = seq.reshape(bb * L, DIM)
    k = jnp.dot(seq2, wk_ref[...], preferred_element_type=jnp.float32) + bk_ref[...]
    v = jnp.dot(seq2, wv_ref[...], preferred_element_type=jnp.float32) + bv_ref[...]
    k3 = k.reshape(bb, L, DIM)
    v3 = v.reshape(bb, L, DIM)
    scale = DIM ** 0.5
    att = jnp.sum(q[:, None, :] * k3, axis=-1) / scale
    att = jax.nn.softmax(att, axis=-1)
    sess_emb = jnp.sum(att[:, :, None] * v3, axis=1)
    nrm = jnp.sqrt(jnp.sum(sess_emb * sess_emb, axis=-1, keepdims=True))
    out_ref[...] = W_K * sess_emb / jnp.maximum(nrm, 1e-12)


def _attention_select(seq_hidden, Wq, bq, Wk, bk, Wv, bv):
    bblk = 128
    grid = (B // bblk,)
    return pl.pallas_call(
        _attn_body,
        grid=grid,
        in_specs=[
            pl.BlockSpec((bblk, L, DIM), lambda i: (i, 0, 0)),
            pl.BlockSpec((DIM, DIM), lambda i: (0, 0)),
            pl.BlockSpec((1, DIM), lambda i: (0, 0)),
            pl.BlockSpec((DIM, DIM), lambda i: (0, 0)),
            pl.BlockSpec((1, DIM), lambda i: (0, 0)),
            pl.BlockSpec((DIM, DIM), lambda i: (0, 0)),
            pl.BlockSpec((1, DIM), lambda i: (0, 0)),
        ],
        out_specs=pl.BlockSpec((bblk, DIM), lambda i: (i, 0)),
        out_shape=jax.ShapeDtypeStruct((B, DIM), jnp.float32),
    )(seq_hidden, Wq, bq.reshape(1, DIM), Wk, bk.reshape(1, DIM),
      Wv, bv.reshape(1, DIM))


def _scores_body(sel_ref, emb_ref, out_ref):
    emb = emb_ref[...]
    nrm = jnp.sqrt(jnp.sum(emb * emb, axis=-1, keepdims=True))
    norm = emb / jnp.maximum(nrm, 1e-12)
    out_ref[...] = jax.lax.dot_general(
        sel_ref[...], norm,
        dimension_numbers=(((1,), (1,)), ((), ())),
        preferred_element_type=jnp.float32)


def _scores(select, emb_pad):
    bblk, nblk = 256, 1280
    grid = (B // bblk, N_PAD // nblk)
    return pl.pallas_call(
        _scores_body,
        grid=grid,
        in_specs=[
            pl.BlockSpec((bblk, DIM), lambda i, j: (i, 0)),
            pl.BlockSpec((nblk, DIM), lambda i, j: (j, 0)),
        ],
        out_specs=pl.BlockSpec((bblk, nblk), lambda i, j: (i, j)),
        out_shape=jax.ShapeDtypeStruct((B, N_PAD), jnp.float32),
    )(select, emb_pad)


def kernel(items, inputs, alias_inputs, edge_index, edge_weight, item_table,
           Wq, bq, Wk, bk, Wv, bv):
    x_pad = jnp.zeros((N_PAD, DIM), jnp.float32).at[:N].set(item_table)
    x_st = jnp.concatenate([x_pad[:, :DH], x_pad[:, DH:]], axis=0)

    pad_e = E_PAD - E
    src = jnp.concatenate([edge_index[1], jnp.zeros((pad_e,), jnp.int32)])
    dst = jnp.concatenate([edge_index[0], jnp.zeros((pad_e,), jnp.int32)])
    w = jnp.concatenate([edge_weight, jnp.zeros((pad_e,), jnp.float32)])
    srcm = src.reshape(E_PAD // C, C)
    dstm = dst.reshape(E_PAD // C, C)
    wm = w

    items_p = jnp.zeros((B, 64), jnp.int32).at[:, :L].set(items).reshape(-1)
    alias_p = jnp.zeros((B, 64), jnp.int32).at[:, :L].set(alias_inputs)

    emb_st, _h1, seq_st = _graph_conv_sc(x_st, srcm, dstm, wm, items_p, alias_p)

    emb_pad = jnp.concatenate([emb_st[:N_PAD], emb_st[N_PAD:]], axis=1)
    seq4 = seq_st.reshape(2, B, 64, DH)
    seq_hidden = jnp.concatenate([seq4[0, :, :L], seq4[1, :, :L]], axis=2)

    select = _attention_select(seq_hidden, Wq, bq, Wk, bk, Wv, bv)
    scores = _scores(select, emb_pad)
    return scores[:, :N]


# speedup vs baseline: 3.3282x; 1.1055x over previous
import functools

import jax
import jax.numpy as jnp
from jax import lax
from jax.experimental import pallas as pl
from jax.experimental.pallas import tpu as pltpu
from jax.experimental.pallas import tpu_sc as plsc

DIM = 128
DH = DIM // 2
NUM_NODE = 10000
N = NUM_NODE + 1
LAYERS = 2
W_K = 12.0
B = 1024
L = 50
E = 320000

N_PAD = 10240
E_PAD = 327680
C = 128
NCH = 160
G = 40
NG = NCH // G
NROW = N_PAD // 16
SESS_PT = B // 16


def _sc_body(x_st, srcm, dstm, wm, items_p, alias_p,
             emb_st, h1_st, seq_st,
             src_v, dst_v, w_v, rows0, rows1, rows2,
             items_v, alias_v, idx_v, srow0, srow1,
             acc,
             gsem0, gsem1):
    c = lax.axis_index("c")
    s = lax.axis_index("s")
    z16 = jnp.zeros((16,), jnp.float32)

    def _zero_row(i, _):
        for k in range(DH // 16):
            rows0[i, pl.ds(k * 16, 16)] = z16
        return 0
    lax.fori_loop(0, C, _zero_row, 0)

    def _zero_acc():
        for k in range(NROW // C):
            pltpu.sync_copy(rows0, acc.at[pl.ds(s * NROW + k * C, C)])
    _zero_acc()

    chunk0 = s * NCH
    coff = c * N_PAD

    plsc.subcore_barrier()

    def _gather_start(table, j, buf, sem):
        return pltpu.async_copy(table.at[src_v.at[j]], buf, sem)

    def _gather_wait(table, buf, sem):
        pltpu.make_async_copy(table.at[pl.ds(0, C)], buf, sem).wait()

    def _scale(buf, lj):
        def _grp(g, _):
            for e16 in range(16):
                ei = g * 16 + e16
                wv = plsc.load_gather(
                    w_v, [jnp.full((16,), lj * C + ei, jnp.int32)])
                for k in range(DH // 16):
                    sl = pl.ds(k * 16, 16)
                    buf[ei, sl] = buf[ei, sl] * wv
            return 0
        lax.fori_loop(0, C // 16, _grp, 0)

    def _layer(table):
        for grp in range(NG):
            g0 = chunk0 + grp * G
            pltpu.sync_copy(srcm.at[pl.ds(g0, G)], src_v)
            pltpu.sync_copy(dstm.at[pl.ds(g0, G)], dst_v)
            pltpu.sync_copy(wm.at[pl.ds(g0 * C, G * C)], w_v)

            def _offs_row(i, _):
                for k in range(C // 16):
                    sl = pl.ds(k * 16, 16)
                    src_v[i, sl] = src_v[i, sl] + coff
                return 0
            lax.fori_loop(0, G, _offs_row, 0)

            _gather_start(table, 0, rows0, gsem0)

            def _pair(i, _):
                j0 = 2 * i
                j1 = 2 * i + 1
                _gather_wait(table, rows0, gsem0)
                _gather_start(table, j1, rows1, gsem1)
                _scale(rows0, j0)
                pltpu.sync_copy(rows0, acc.at[dst_v.at[j0]], add=True)
                _gather_wait(table, rows1, gsem1)

                @pl.when(i < G // 2 - 1)
                def _():
                    _gather_start(table, j1 + 1, rows0, gsem0)
                _scale(rows1, j1)
                pltpu.sync_copy(rows1, acc.at[dst_v.at[j1]], add=True)
                return 0
            lax.fori_loop(0, G // 2, _pair, 0)

    _layer(x_st)
    plsc.subcore_barrier()

    for k in range(NROW // C):
        r0 = s * NROW + k * C
        pltpu.sync_copy(acc.at[pl.ds(r0, C)], rows1)
        pltpu.sync_copy(rows1, h1_st.at[pl.ds(coff + r0, C)])
    lax.fori_loop(0, C, _zero_row, 0)
    _zero_acc()
    plsc.subcore_barrier()

    _layer(h1_st)
    plsc.subcore_barrier()

    third = jnp.float32(1.0 / 3.0)
    for k in range(NROW // C):
        r0 = s * NROW + k * C
        pltpu.sync_copy(x_st.at[pl.ds(coff + r0, C)], rows0)
        pltpu.sync_copy(h1_st.at[pl.ds(coff + r0, C)], rows1)
        pltpu.sync_copy(acc.at[pl.ds(r0, C)], rows2)

        def _comb(i, _):
            for kk in range(DH // 16):
                sl = pl.ds(kk * 16, 16)
                rows0[i, sl] = (rows0[i, sl] + rows1[i, sl]
                                + rows2[i, sl]) * third
            return 0
        lax.fori_loop(0, C, _comb, 0)
        pltpu.sync_copy(rows0, emb_st.at[pl.ds(coff + r0, C)])
    plsc.subcore_barrier()

    sess0 = s * SESS_PT
    pltpu.sync_copy(items_p.at[pl.ds(sess0 * 64, SESS_PT * 64)], items_v)
    pltpu.sync_copy(alias_p.at[pl.ds(sess0, SESS_PT)], alias_v)

    def _mkidx(i, _):
        for g in range(4):
            sl = pl.ds(g * 16, 16)
            av = alias_v[i, sl]
            cb = plsc.load_gather(items_v, [av + i * 64])
            idx_v[i, sl] = cb + coff
        return 0
    lax.fori_loop(0, SESS_PT, _mkidx, 0)

    soff = c * (B * 64)

    def _sgather_start(i, buf, sem):
        return pltpu.async_copy(emb_st.at[idx_v.at[i]], buf, sem)

    def _sgather_wait(buf, sem):
        pltpu.make_async_copy(emb_st.at[pl.ds(0, 64)], buf, sem).wait()

    _sgather_start(0, srow0, gsem0)

    def _sess_pair(i, _):
        i0 = 2 * i
        i1 = 2 * i + 1
        _sgather_wait(srow0, gsem0)
        _sgather_start(i1, srow1, gsem1)
        pltpu.sync_copy(srow0,
                        seq_st.at[pl.ds(soff + (sess0 + i0) * 64, 64)])
        _sgather_wait(srow1, gsem1)

        @pl.when(i < SESS_PT // 2 - 1)
        def _():
            _sgather_start(i1 + 1, srow0, gsem0)
        pltpu.sync_copy(srow1,
                        seq_st.at[pl.ds(soff + (sess0 + i1) * 64, 64)])
        return 0
    lax.fori_loop(0, SESS_PT // 2, _sess_pair, 0)


@jax.jit
def _graph_conv_sc(x_st, srcm, dstm, wm, items_p, alias_p):
    mesh = plsc.VectorSubcoreMesh(core_axis_name="c", subcore_axis_name="s")
    f = pl.kernel(
        _sc_body,
        out_type=(
            jax.ShapeDtypeStruct((2 * N_PAD, DH), jnp.float32),
            jax.ShapeDtypeStruct((2 * N_PAD, DH), jnp.float32),
            jax.ShapeDtypeStruct((2 * B * 64, DH), jnp.float32),
        ),
        mesh=mesh,
        compiler_params=pltpu.CompilerParams(
            needs_layout_passes=False, use_tc_tiling_on_sc=False),
        scratch_types=[
            pltpu.VMEM((G, C), jnp.int32),
            pltpu.VMEM((G, C), jnp.int32),
            pltpu.VMEM((G * C,), jnp.float32),
            pltpu.VMEM((C, DH), jnp.float32),
            pltpu.VMEM((C, DH), jnp.float32),
            pltpu.VMEM((C, DH), jnp.float32),
            pltpu.VMEM((SESS_PT * 64,), jnp.int32),
            pltpu.VMEM((SESS_PT, 64), jnp.int32),
            pltpu.VMEM((SESS_PT, 64), jnp.int32),
            pltpu.VMEM((64, DH), jnp.float32),
            pltpu.VMEM((64, DH), jnp.float32),
            pltpu.VMEM_SHARED((N_PAD, DH), jnp.float32),
            pltpu.SemaphoreType.DMA,
            pltpu.SemaphoreType.DMA,
        ],
    )
    return f(x_st, srcm, dstm, wm, items_p, alias_p)


def _attn_body(lo_ref, hi_ref, wq_ref, bq_ref, wk_ref, bk_ref, wv_ref, bv_ref,
               out_ref):
    seq = jnp.concatenate([lo_ref[0], hi_ref[0]], axis=-1)
    lmask = lax.broadcasted_iota(jnp.int32, (1, 64, 1), 1) < L
    seq = jnp.where(lmask, seq, 0.0)
    bb = seq.shape[0]
    sess = jnp.sum(seq, axis=1)
    q = jnp.dot(sess, wq_ref[...], preferred_element_type=jnp.float32) + bq_ref[...]
    seq2 = seq.reshape(bb * 64, DIM)
    k = jnp.dot(seq2, wk_ref[...], preferred_element_type=jnp.float32) + bk_ref[...]
    v = jnp.dot(seq2, wv_ref[...], preferred_element_type=jnp.float32) + bv_ref[...]
    k3 = k.reshape(bb, 64, DIM)
    v3 = v.reshape(bb, 64, DIM)
    scale = DIM ** 0.5
    att = jnp.sum(q[:, None, :] * k3, axis=-1) / scale
    att = jnp.where(lax.broadcasted_iota(jnp.int32, (1, 64), 1) < L,
                    att, -1e30)
    att = jax.nn.softmax(att, axis=-1)
    sess_emb = jnp.sum(att[:, :, None] * v3, axis=1)
    nrm = jnp.sqrt(jnp.sum(sess_emb * sess_emb, axis=-1, keepdims=True))
    out_ref[...] = W_K * sess_emb / jnp.maximum(nrm, 1e-12)


def _attention_select(seq_st, Wq, bq, Wk, bk, Wv, bv):
    bblk = 128
    grid = (B // bblk,)
    seq4 = seq_st.reshape(2, B, 64, DH)
    return pl.pallas_call(
        _attn_body,
        grid=grid,
        in_specs=[
            pl.BlockSpec((1, bblk, 64, DH), lambda i: (0, i, 0, 0)),
            pl.BlockSpec((1, bblk, 64, DH), lambda i: (1, i, 0, 0)),
            pl.BlockSpec((DIM, DIM), lambda i: (0, 0)),
            pl.BlockSpec((1, DIM), lambda i: (0, 0)),
            pl.BlockSpec((DIM, DIM), lambda i: (0, 0)),
            pl.BlockSpec((1, DIM), lambda i: (0, 0)),
            pl.BlockSpec((DIM, DIM), lambda i: (0, 0)),
            pl.BlockSpec((1, DIM), lambda i: (0, 0)),
        ],
        out_specs=pl.BlockSpec((bblk, DIM), lambda i: (i, 0)),
        out_shape=jax.ShapeDtypeStruct((B, DIM), jnp.float32),
    )(seq4, seq4, Wq, bq.reshape(1, DIM), Wk, bk.reshape(1, DIM),
      Wv, bv.reshape(1, DIM))


def _scores_body(sel_ref, lo_ref, hi_ref, out_ref):
    emb = jnp.concatenate([lo_ref[0], hi_ref[0]], axis=-1)
    nrm = jnp.sqrt(jnp.sum(emb * emb, axis=-1, keepdims=True))
    norm = emb / jnp.maximum(nrm, 1e-12)
    out_ref[...] = jax.lax.dot_general(
        sel_ref[...], norm,
        dimension_numbers=(((1,), (1,)), ((), ())),
        preferred_element_type=jnp.float32)


def _scores(select, emb_st):
    bblk, nblk = 256, 1280
    grid = (B // bblk, N_PAD // nblk)
    emb3 = emb_st.reshape(2, N_PAD, DH)
    return pl.pallas_call(
        _scores_body,
        grid=grid,
        in_specs=[
            pl.BlockSpec((bblk, DIM), lambda i, j: (i, 0)),
            pl.BlockSpec((1, nblk, DH), lambda i, j: (0, j, 0)),
            pl.BlockSpec((1, nblk, DH), lambda i, j: (1, j, 0)),
        ],
        out_specs=pl.BlockSpec((bblk, nblk), lambda i, j: (i, j)),
        out_shape=jax.ShapeDtypeStruct((B, N), jnp.float32),
    )(select, emb3, emb3)


def kernel(items, inputs, alias_inputs, edge_index, edge_weight, item_table,
           Wq, bq, Wk, bk, Wv, bv):
    x_st = (jnp.zeros((2 * N_PAD, DH), jnp.float32)
            .at[:N].set(item_table[:, :DH])
            .at[N_PAD:N_PAD + N].set(item_table[:, DH:]))

    pad_e = E_PAD - E
    src = jnp.concatenate([edge_index[1], jnp.zeros((pad_e,), jnp.int32)])
    dst = jnp.concatenate([edge_index[0], jnp.zeros((pad_e,), jnp.int32)])
    w = jnp.concatenate([edge_weight, jnp.zeros((pad_e,), jnp.float32)])
    srcm = src.reshape(E_PAD // C, C)
    dstm = dst.reshape(E_PAD // C, C)
    wm = w

    items_p = jnp.zeros((B, 64), jnp.int32).at[:, :L].set(items).reshape(-1)
    alias_p = jnp.zeros((B, 64), jnp.int32).at[:, :L].set(alias_inputs)

    emb_st, _h1, seq_st = _graph_conv_sc(x_st, srcm, dstm, wm, items_p, alias_p)

    select = _attention_select(seq_st, Wq, bq, Wk, bk, Wv, bv)
    return _scores(select, emb_st)


# speedup vs baseline: 3.6894x; 1.1085x over previous
import functools

import jax
import jax.numpy as jnp
from jax import lax
from jax.experimental import pallas as pl
from jax.experimental.pallas import tpu as pltpu
from jax.experimental.pallas import tpu_sc as plsc

DIM = 128
DH = DIM // 2
NUM_NODE = 10000
N = NUM_NODE + 1
LAYERS = 2
W_K = 12.0
B = 1024
L = 50
E = 320000

N_PAD = 10240
E_PAD = 327680
C = 128
NCH = 160
G = 40
NG = NCH // G
NROW = N_PAD // 16
SESS_PT = B // 16


def _sc_body(x_st, srcm, dstm, wm, items_p, alias_p,
             emb_st, h1_st, seq_st,
             src_v, dst_v, w_v, rows0, rows1, rows2, rows3,
             items_v, alias_v, idx_v, srow0, srow1,
             acc,
             gsem0, gsem1, gsem2, gsem3,
             ssem0, ssem1, ssem2, ssem3):
    c = lax.axis_index("c")
    s = lax.axis_index("s")
    z16 = jnp.zeros((16,), jnp.float32)

    def _zero_row(i, _):
        for k in range(DH // 16):
            rows0[i, pl.ds(k * 16, 16)] = z16
        return 0
    lax.fori_loop(0, C, _zero_row, 0)

    def _zero_acc():
        for k in range(NROW // C):
            pltpu.sync_copy(rows0, acc.at[pl.ds(s * NROW + k * C, C)])
    _zero_acc()

    chunk0 = s * NCH
    coff = c * N_PAD

    plsc.subcore_barrier()

    def _gather_start(table, j, buf, sem):
        return pltpu.async_copy(table.at[src_v.at[j]], buf, sem)

    def _gather_wait(table, buf, sem):
        pltpu.make_async_copy(table.at[pl.ds(0, C)], buf, sem).wait()

    def _scale(buf, lj):
        def _grp(g, _):
            for e16 in range(16):
                ei = g * 16 + e16
                wv = plsc.load_gather(
                    w_v, [jnp.full((16,), lj * C + ei, jnp.int32)])
                for k in range(DH // 16):
                    sl = pl.ds(k * 16, 16)
                    buf[ei, sl] = buf[ei, sl] * wv
            return 0
        lax.fori_loop(0, C // 16, _grp, 0)

    def _scatter_start(buf, j, sem):
        pltpu.async_copy(buf, acc.at[dst_v.at[j]], sem, add=True)

    def _scatter_wait(buf, sem):
        pltpu.make_async_copy(buf, acc.at[pl.ds(0, C)], sem).wait()

    bufs = (rows0, rows1, rows2, rows3)
    gsems = (gsem0, gsem1, gsem2, gsem3)
    ssems = (ssem0, ssem1, ssem2, ssem3)

    def _layer(table):
        def _group(grp, _):
            g0 = chunk0 + grp * G
            pltpu.sync_copy(srcm.at[pl.ds(g0, G)], src_v)
            pltpu.sync_copy(dstm.at[pl.ds(g0, G)], dst_v)
            pltpu.sync_copy(wm.at[pl.ds(g0 * C, G * C)], w_v)

            def _offs_row(i, _):
                for k in range(C // 16):
                    sl = pl.ds(k * 16, 16)
                    src_v[i, sl] = src_v[i, sl] + coff
                return 0
            lax.fori_loop(0, G, _offs_row, 0)

            _gather_start(table, 0, bufs[0], gsems[0])
            _gather_start(table, 1, bufs[1], gsems[1])

            def _quad(i, _):
                for k in range(4):
                    j = 4 * i + k
                    k2 = (k + 2) % 4
                    _gather_wait(table, bufs[k], gsems[k])
                    _scale(bufs[k], j)
                    _scatter_start(bufs[k], j, ssems[k])
                    if k < 2:
                        @pl.when(i > 0)
                        def _():
                            _scatter_wait(bufs[k2], ssems[k2])
                        _gather_start(table, j + 2, bufs[k2], gsems[k2])
                    else:
                        _scatter_wait(bufs[k2], ssems[k2])

                        @pl.when(j + 2 < G)
                        def _():
                            _gather_start(table, j + 2, bufs[k2], gsems[k2])
                return 0
            lax.fori_loop(0, G // 4, _quad, 0)
            _scatter_wait(bufs[2], ssems[2])
            _scatter_wait(bufs[3], ssems[3])
            return 0
        lax.fori_loop(0, NG, _group, 0)

    _layer(x_st)
    plsc.subcore_barrier()

    for k in range(NROW // C):
        r0 = s * NROW + k * C
        pltpu.sync_copy(acc.at[pl.ds(r0, C)], rows1)
        pltpu.sync_copy(rows1, h1_st.at[pl.ds(coff + r0, C)])
    lax.fori_loop(0, C, _zero_row, 0)
    _zero_acc()
    plsc.subcore_barrier()

    _layer(h1_st)
    plsc.subcore_barrier()

    third = jnp.float32(1.0 / 3.0)
    for k in range(NROW // C):
        r0 = s * NROW + k * C
        pltpu.sync_copy(x_st.at[pl.ds(coff + r0, C)], rows0)
        pltpu.sync_copy(h1_st.at[pl.ds(coff + r0, C)], rows1)
        pltpu.sync_copy(acc.at[pl.ds(r0, C)], rows2)

        def _comb(i, _):
            for kk in range(DH // 16):
                sl = pl.ds(kk * 16, 16)
                rows0[i, sl] = (rows0[i, sl] + rows1[i, sl]
                                + rows2[i, sl]) * third
            return 0
        lax.fori_loop(0, C, _comb, 0)
        pltpu.sync_copy(rows0, emb_st.at[pl.ds(coff + r0, C)])
    plsc.subcore_barrier()

    sess0 = s * SESS_PT
    pltpu.sync_copy(items_p.at[pl.ds(sess0 * 64, SESS_PT * 64)], items_v)
    pltpu.sync_copy(alias_p.at[pl.ds(sess0, SESS_PT)], alias_v)

    def _mkidx(i, _):
        for g in range(4):
            sl = pl.ds(g * 16, 16)
            av = alias_v[i, sl]
            cb = plsc.load_gather(items_v, [av + i * 64])
            idx_v[i, sl] = cb + coff
        return 0
    lax.fori_loop(0, SESS_PT, _mkidx, 0)

    soff = c * (B * 64)

    def _sgather_start(i, buf, sem):
        return pltpu.async_copy(emb_st.at[idx_v.at[i]], buf, sem)

    def _sgather_wait(buf, sem):
        pltpu.make_async_copy(emb_st.at[pl.ds(0, 64)], buf, sem).wait()

    _sgather_start(0, srow0, gsem0)

    def _sess_pair(i, _):
        i0 = 2 * i
        i1 = 2 * i + 1
        _sgather_wait(srow0, gsem0)
        _sgather_start(i1, srow1, gsem1)
        pltpu.sync_copy(srow0,
                        seq_st.at[pl.ds(soff + (sess0 + i0) * 64, 64)])
        _sgather_wait(srow1, gsem1)

        @pl.when(i < SESS_PT // 2 - 1)
        def _():
            _sgather_start(i1 + 1, srow0, gsem0)
        pltpu.sync_copy(srow1,
                        seq_st.at[pl.ds(soff + (sess0 + i1) * 64, 64)])
        return 0
    lax.fori_loop(0, SESS_PT // 2, _sess_pair, 0)


@jax.jit
def _graph_conv_sc(x_st, srcm, dstm, wm, items_p, alias_p):
    mesh = plsc.VectorSubcoreMesh(core_axis_name="c", subcore_axis_name="s")
    f = pl.kernel(
        _sc_body,
        out_type=(
            jax.ShapeDtypeStruct((2 * N_PAD, DH), jnp.float32),
            jax.ShapeDtypeStruct((2 * N_PAD, DH), jnp.float32),
            jax.ShapeDtypeStruct((2 * B * 64, DH), jnp.float32),
        ),
        mesh=mesh,
        compiler_params=pltpu.CompilerParams(
            needs_layout_passes=False, use_tc_tiling_on_sc=False),
        scratch_types=[
            pltpu.VMEM((G, C), jnp.int32),
            pltpu.VMEM((G, C), jnp.int32),
            pltpu.VMEM((G * C,), jnp.float32),
            pltpu.VMEM((C, DH), jnp.float32),
            pltpu.VMEM((C, DH), jnp.float32),
            pltpu.VMEM((C, DH), jnp.float32),
            pltpu.VMEM((C, DH), jnp.float32),
            pltpu.VMEM((SESS_PT * 64,), jnp.int32),
            pltpu.VMEM((SESS_PT, 64), jnp.int32),
            pltpu.VMEM((SESS_PT, 64), jnp.int32),
            pltpu.VMEM((64, DH), jnp.float32),
            pltpu.VMEM((64, DH), jnp.float32),
            pltpu.VMEM_SHARED((N_PAD, DH), jnp.float32),
            pltpu.SemaphoreType.DMA,
            pltpu.SemaphoreType.DMA,
            pltpu.SemaphoreType.DMA,
            pltpu.SemaphoreType.DMA,
            pltpu.SemaphoreType.DMA,
            pltpu.SemaphoreType.DMA,
            pltpu.SemaphoreType.DMA,
            pltpu.SemaphoreType.DMA,
        ],
    )
    return f(x_st, srcm, dstm, wm, items_p, alias_p)


def _attn_body(lo_ref, hi_ref, wq_ref, bq_ref, wk_ref, bk_ref, wv_ref, bv_ref,
               out_ref):
    seq = jnp.concatenate([lo_ref[0], hi_ref[0]], axis=-1)
    lmask = lax.broadcasted_iota(jnp.int32, (1, 64, 1), 1) < L
    seq = jnp.where(lmask, seq, 0.0)
    bb = seq.shape[0]
    sess = jnp.sum(seq, axis=1)
    q = jnp.dot(sess, wq_ref[...], preferred_element_type=jnp.float32) + bq_ref[...]
    seq2 = seq.reshape(bb * 64, DIM)
    k = jnp.dot(seq2, wk_ref[...], preferred_element_type=jnp.float32) + bk_ref[...]
    v = jnp.dot(seq2, wv_ref[...], preferred_element_type=jnp.float32) + bv_ref[...]
    k3 = k.reshape(bb, 64, DIM)
    v3 = v.reshape(bb, 64, DIM)
    scale = DIM ** 0.5
    att = jnp.sum(q[:, None, :] * k3, axis=-1) / scale
    att = jnp.where(lax.broadcasted_iota(jnp.int32, (1, 64), 1) < L,
                    att, -1e30)
    att = jax.nn.softmax(att, axis=-1)
    sess_emb = jnp.sum(att[:, :, None] * v3, axis=1)
    nrm = jnp.sqrt(jnp.sum(sess_emb * sess_emb, axis=-1, keepdims=True))
    out_ref[...] = W_K * sess_emb / jnp.maximum(nrm, 1e-12)


def _attention_select(seq_st, Wq, bq, Wk, bk, Wv, bv):
    bblk = 128
    grid = (B // bblk,)
    seq4 = seq_st.reshape(2, B, 64, DH)
    return pl.pallas_call(
        _attn_body,
        grid=grid,
        in_specs=[
            pl.BlockSpec((1, bblk, 64, DH), lambda i: (0, i, 0, 0)),
            pl.BlockSpec((1, bblk, 64, DH), lambda i: (1, i, 0, 0)),
            pl.BlockSpec((DIM, DIM), lambda i: (0, 0)),
            pl.BlockSpec((1, DIM), lambda i: (0, 0)),
            pl.BlockSpec((DIM, DIM), lambda i: (0, 0)),
            pl.BlockSpec((1, DIM), lambda i: (0, 0)),
            pl.BlockSpec((DIM, DIM), lambda i: (0, 0)),
            pl.BlockSpec((1, DIM), lambda i: (0, 0)),
        ],
        out_specs=pl.BlockSpec((bblk, DIM), lambda i: (i, 0)),
        out_shape=jax.ShapeDtypeStruct((B, DIM), jnp.float32),
    )(seq4, seq4, Wq, bq.reshape(1, DIM), Wk, bk.reshape(1, DIM),
      Wv, bv.reshape(1, DIM))


def _scores_body(sel_ref, lo_ref, hi_ref, out_ref):
    emb = jnp.concatenate([lo_ref[0], hi_ref[0]], axis=-1)
    nrm = jnp.sqrt(jnp.sum(emb * emb, axis=-1, keepdims=True))
    norm = emb / jnp.maximum(nrm, 1e-12)
    out_ref[...] = jax.lax.dot_general(
        sel_ref[...], norm,
        dimension_numbers=(((1,), (1,)), ((), ())),
        preferred_element_type=jnp.float32)


def _scores(select, emb_st):
    bblk, nblk = 256, 1280
    grid = (B // bblk, N_PAD // nblk)
    emb3 = emb_st.reshape(2, N_PAD, DH)
    return pl.pallas_call(
        _scores_body,
        grid=grid,
        in_specs=[
            pl.BlockSpec((bblk, DIM), lambda i, j: (i, 0)),
            pl.BlockSpec((1, nblk, DH), lambda i, j: (0, j, 0)),
            pl.BlockSpec((1, nblk, DH), lambda i, j: (1, j, 0)),
        ],
        out_specs=pl.BlockSpec((bblk, nblk), lambda i, j: (i, j)),
        out_shape=jax.ShapeDtypeStruct((B, N), jnp.float32),
    )(select, emb3, emb3)


def kernel(items, inputs, alias_inputs, edge_index, edge_weight, item_table,
           Wq, bq, Wk, bk, Wv, bv):
    x_st = (jnp.zeros((2 * N_PAD, DH), jnp.float32)
            .at[:N].set(item_table[:, :DH])
            .at[N_PAD:N_PAD + N].set(item_table[:, DH:]))

    pad_e = E_PAD - E
    src = jnp.concatenate([edge_index[1], jnp.zeros((pad_e,), jnp.int32)])
    dst = jnp.concatenate([edge_index[0], jnp.zeros((pad_e,), jnp.int32)])
    w = jnp.concatenate([edge_weight, jnp.zeros((pad_e,), jnp.float32)])
    srcm = src.reshape(E_PAD // C, C)
    dstm = dst.reshape(E_PAD // C, C)
    wm = w

    items_p = jnp.zeros((B, 64), jnp.int32).at[:, :L].set(items).reshape(-1)
    alias_p = jnp.zeros((B, 64), jnp.int32).at[:, :L].set(alias_inputs)

    emb_st, _h1, seq_st = _graph_conv_sc(x_st, srcm, dstm, wm, items_p, alias_p)

    select = _attention_select(seq_st, Wq, bq, Wk, bk, Wv, bv)
    return _scores(select, emb_st)


# speedup vs baseline: 3.6920x; 1.0007x over previous
import functools

import jax
import jax.numpy as jnp
from jax import lax
from jax.experimental import pallas as pl
from jax.experimental.pallas import tpu as pltpu
from jax.experimental.pallas import tpu_sc as plsc

DIM = 128
DH = DIM // 2
NUM_NODE = 10000
N = NUM_NODE + 1
LAYERS = 2
W_K = 12.0
B = 1024
L = 50
E = 320000

N_PAD = 10240
E_PAD = 327680
C = 128
NCH = 160
G = 40
NG = NCH // G
NROW = N_PAD // 16
SESS_PT = B // 16


def _sc_body(x_st, srcm, dstm, wm, items_p, alias_p,
             emb_st, h1_st, seq_st,
             src_v, dst_v, w_v, rows0, rows1, rows2, rows3,
             items_v, alias_v, idx_v, srow0, srow1,
             acc,
             gsem0, gsem1, gsem2, gsem3,
             ssem0, ssem1, ssem2, ssem3):
    c = lax.axis_index("c")
    s = lax.axis_index("s")
    z16 = jnp.zeros((16,), jnp.float32)

    def _zero_row(i, _):
        for k in range(DH // 16):
            rows0[i, pl.ds(k * 16, 16)] = z16
        return 0
    lax.fori_loop(0, C, _zero_row, 0)

    def _zero_acc():
        for k in range(NROW // C):
            pltpu.sync_copy(rows0, acc.at[pl.ds(s * NROW + k * C, C)])
    _zero_acc()

    chunk0 = s * NCH
    coff = c * N_PAD

    plsc.subcore_barrier()

    def _gather_start(table, j, buf, sem):
        return pltpu.async_copy(table.at[src_v.at[j]], buf, sem)

    def _gather_wait(table, buf, sem):
        pltpu.make_async_copy(table.at[pl.ds(0, C)], buf, sem).wait()

    def _scale(buf, lj):
        def _grp(g, _):
            for e16 in range(16):
                ei = g * 16 + e16
                wv = plsc.load_gather(
                    w_v, [jnp.full((16,), lj * C + ei, jnp.int32)])
                for k in range(DH // 16):
                    sl = pl.ds(k * 16, 16)
                    buf[ei, sl] = buf[ei, sl] * wv
            return 0
        lax.fori_loop(0, C // 16, _grp, 0)

    def _scatter_start(buf, j, sem):
        pltpu.async_copy(buf, acc.at[dst_v.at[j]], sem, add=True)

    def _scatter_wait(buf, sem):
        pltpu.make_async_copy(buf, acc.at[pl.ds(0, C)], sem).wait()

    bufs = (rows0, rows1, rows2, rows3)
    gsems = (gsem0, gsem1, gsem2, gsem3)
    ssems = (ssem0, ssem1, ssem2, ssem3)

    def _layer(table):
        def _group(grp, _):
            g0 = chunk0 + grp * G
            pltpu.sync_copy(srcm.at[pl.ds(g0, G)], src_v)
            pltpu.sync_copy(dstm.at[pl.ds(g0, G)], dst_v)
            pltpu.sync_copy(wm.at[pl.ds(g0 * C, G * C)], w_v)

            def _offs_row(i, _):
                for k in range(C // 16):
                    sl = pl.ds(k * 16, 16)
                    src_v[i, sl] = src_v[i, sl] + coff
                return 0
            lax.fori_loop(0, G, _offs_row, 0)

            _gather_start(table, 0, bufs[0], gsems[0])
            _gather_start(table, 1, bufs[1], gsems[1])

            def _quad(i, _):
                for k in range(4):
                    j = 4 * i + k
                    k2 = (k + 2) % 4
                    _gather_wait(table, bufs[k], gsems[k])
                    _scale(bufs[k], j)
                    _scatter_start(bufs[k], j, ssems[k])
                    if k < 2:
                        @pl.when(i > 0)
                        def _():
                            _scatter_wait(bufs[k2], ssems[k2])
                        _gather_start(table, j + 2, bufs[k2], gsems[k2])
                    else:
                        _scatter_wait(bufs[k2], ssems[k2])

                        @pl.when(j + 2 < G)
                        def _():
                            _gather_start(table, j + 2, bufs[k2], gsems[k2])
                return 0
            lax.fori_loop(0, G // 4, _quad, 0)
            _scatter_wait(bufs[2], ssems[2])
            _scatter_wait(bufs[3], ssems[3])
            return 0
        lax.fori_loop(0, NG, _group, 0)

    _layer(x_st)
    plsc.subcore_barrier()

    for k in range(NROW // C):
        r0 = s * NROW + k * C
        pltpu.sync_copy(acc.at[pl.ds(r0, C)], rows1)
        pltpu.sync_copy(rows1, h1_st.at[pl.ds(coff + r0, C)])
    lax.fori_loop(0, C, _zero_row, 0)
    _zero_acc()
    plsc.subcore_barrier()

    _layer(h1_st)
    plsc.subcore_barrier()

    third = jnp.float32(1.0 / 3.0)
    for k in range(NROW // C):
        r0 = s * NROW + k * C
        pltpu.sync_copy(x_st.at[pl.ds(coff + r0, C)], rows0)
        pltpu.sync_copy(h1_st.at[pl.ds(coff + r0, C)], rows1)
        pltpu.sync_copy(acc.at[pl.ds(r0, C)], rows2)

        def _comb(i, _):
            for kk in range(DH // 16):
                sl = pl.ds(kk * 16, 16)
                rows0[i, sl] = (rows0[i, sl] + rows1[i, sl]
                                + rows2[i, sl]) * third
            return 0
        lax.fori_loop(0, C, _comb, 0)
        pltpu.sync_copy(rows0, emb_st.at[pl.ds(coff + r0, C)])
    plsc.subcore_barrier()

    sess0 = s * SESS_PT
    pltpu.sync_copy(items_p.at[pl.ds(sess0 * 64, SESS_PT * 64)], items_v)
    pltpu.sync_copy(alias_p.at[pl.ds(sess0, SESS_PT)], alias_v)

    def _mkidx(i, _):
        for g in range(4):
            sl = pl.ds(g * 16, 16)
            av = alias_v[i, sl]
            cb = plsc.load_gather(items_v, [av + i * 64])
            idx_v[i, sl] = cb + coff
        return 0
    lax.fori_loop(0, SESS_PT, _mkidx, 0)

    soff = c * (B * 64)

    def _sgather_start(i, buf, sem):
        return pltpu.async_copy(emb_st.at[idx_v.at[i]], buf, sem)

    def _sgather_wait(buf, sem):
        pltpu.make_async_copy(emb_st.at[pl.ds(0, 64)], buf, sem).wait()

    _sgather_start(0, srow0, gsem0)

    def _sess_pair(i, _):
        i0 = 2 * i
        i1 = 2 * i + 1
        _sgather_wait(srow0, gsem0)
        _sgather_start(i1, srow1, gsem1)
        pltpu.sync_copy(srow0,
                        seq_st.at[pl.ds(soff + (sess0 + i0) * 64, 64)])
        _sgather_wait(srow1, gsem1)

        @pl.when(i < SESS_PT // 2 - 1)
        def _():
            _sgather_start(i1 + 1, srow0, gsem0)
        pltpu.sync_copy(srow1,
                        seq_st.at[pl.ds(soff + (sess0 + i1) * 64, 64)])
        return 0
    lax.fori_loop(0, SESS_PT // 2, _sess_pair, 0)


@jax.jit
def _graph_conv_sc(x_st, srcm, dstm, wm, items_p, alias_p):
    mesh = plsc.VectorSubcoreMesh(core_axis_name="c", subcore_axis_name="s")
    f = pl.kernel(
        _sc_body,
        out_type=(
            jax.ShapeDtypeStruct((2 * N_PAD, DH), jnp.float32),
            jax.ShapeDtypeStruct((2 * N_PAD, DH), jnp.float32),
            jax.ShapeDtypeStruct((2 * B * 64, DH), jnp.float32),
        ),
        mesh=mesh,
        compiler_params=pltpu.CompilerParams(
            needs_layout_passes=False, use_tc_tiling_on_sc=False),
        scratch_types=[
            pltpu.VMEM((G, C), jnp.int32),
            pltpu.VMEM((G, C), jnp.int32),
            pltpu.VMEM((G * C,), jnp.float32),
            pltpu.VMEM((C, DH), jnp.float32),
            pltpu.VMEM((C, DH), jnp.float32),
            pltpu.VMEM((C, DH), jnp.float32),
            pltpu.VMEM((C, DH), jnp.float32),
            pltpu.VMEM((SESS_PT * 64,), jnp.int32),
            pltpu.VMEM((SESS_PT, 64), jnp.int32),
            pltpu.VMEM((SESS_PT, 64), jnp.int32),
            pltpu.VMEM((64, DH), jnp.float32),
            pltpu.VMEM((64, DH), jnp.float32),
            pltpu.VMEM_SHARED((N_PAD, DH), jnp.float32),
            pltpu.SemaphoreType.DMA,
            pltpu.SemaphoreType.DMA,
            pltpu.SemaphoreType.DMA,
            pltpu.SemaphoreType.DMA,
            pltpu.SemaphoreType.DMA,
            pltpu.SemaphoreType.DMA,
            pltpu.SemaphoreType.DMA,
            pltpu.SemaphoreType.DMA,
        ],
    )
    return f(x_st, srcm, dstm, wm, items_p, alias_p)


def _attn_body(lo_ref, hi_ref, wq_ref, bq_ref, wk_ref, bk_ref, wv_ref, bv_ref,
               out_ref):
    nb = lo_ref.shape[0] // 64
    seq = jnp.concatenate([lo_ref[...].reshape(nb, 64, DH),
                           hi_ref[...].reshape(nb, 64, DH)], axis=-1)
    lmask = lax.broadcasted_iota(jnp.int32, (1, 64, 1), 1) < L
    seq = jnp.where(lmask, seq, 0.0)
    bb = seq.shape[0]
    sess = jnp.sum(seq, axis=1)
    q = jnp.dot(sess, wq_ref[...], preferred_element_type=jnp.float32) + bq_ref[...]
    seq2 = seq.reshape(bb * 64, DIM)
    k = jnp.dot(seq2, wk_ref[...], preferred_element_type=jnp.float32) + bk_ref[...]
    v = jnp.dot(seq2, wv_ref[...], preferred_element_type=jnp.float32) + bv_ref[...]
    k3 = k.reshape(bb, 64, DIM)
    v3 = v.reshape(bb, 64, DIM)
    scale = DIM ** 0.5
    att = jnp.sum(q[:, None, :] * k3, axis=-1) / scale
    att = jnp.where(lax.broadcasted_iota(jnp.int32, (1, 64), 1) < L,
                    att, -1e30)
    att = jax.nn.softmax(att, axis=-1)
    sess_emb = jnp.sum(att[:, :, None] * v3, axis=1)
    nrm = jnp.sqrt(jnp.sum(sess_emb * sess_emb, axis=-1, keepdims=True))
    out_ref[...] = W_K * sess_emb / jnp.maximum(nrm, 1e-12)


def _attention_select(seq_st, Wq, bq, Wk, bk, Wv, bv):
    bblk = 128
    grid = (B // bblk,)
    return pl.pallas_call(
        _attn_body,
        grid=grid,
        in_specs=[
            pl.BlockSpec((bblk * 64, DH), lambda i: (i, 0)),
            pl.BlockSpec((bblk * 64, DH), lambda i: (i + B // bblk, 0)),
            pl.BlockSpec((DIM, DIM), lambda i: (0, 0)),
            pl.BlockSpec((1, DIM), lambda i: (0, 0)),
            pl.BlockSpec((DIM, DIM), lambda i: (0, 0)),
            pl.BlockSpec((1, DIM), lambda i: (0, 0)),
            pl.BlockSpec((DIM, DIM), lambda i: (0, 0)),
            pl.BlockSpec((1, DIM), lambda i: (0, 0)),
        ],
        out_specs=pl.BlockSpec((bblk, DIM), lambda i: (i, 0)),
        out_shape=jax.ShapeDtypeStruct((B, DIM), jnp.float32),
    )(seq_st, seq_st, Wq, bq.reshape(1, DIM), Wk, bk.reshape(1, DIM),
      Wv, bv.reshape(1, DIM))


def _scores_body(sel_ref, lo_ref, hi_ref, out_ref):
    emb = jnp.concatenate([lo_ref[...], hi_ref[...]], axis=-1)
    nrm = jnp.sqrt(jnp.sum(emb * emb, axis=-1, keepdims=True))
    norm = emb / jnp.maximum(nrm, 1e-12)
    out_ref[...] = jax.lax.dot_general(
        sel_ref[...], norm,
        dimension_numbers=(((1,), (1,)), ((), ())),
        preferred_element_type=jnp.float32)


def _scores(select, emb_st):
    bblk, nblk = 256, 1280
    grid = (B // bblk, N_PAD // nblk)
    return pl.pallas_call(
        _scores_body,
        grid=grid,
        in_specs=[
            pl.BlockSpec((bblk, DIM), lambda i, j: (i, 0)),
            pl.BlockSpec((nblk, DH), lambda i, j: (j, 0)),
            pl.BlockSpec((nblk, DH), lambda i, j: (j + N_PAD // nblk, 0)),
        ],
        out_specs=pl.BlockSpec((bblk, nblk), lambda i, j: (i, j)),
        out_shape=jax.ShapeDtypeStruct((B, N), jnp.float32),
    )(select, emb_st, emb_st)


def kernel(items, inputs, alias_inputs, edge_index, edge_weight, item_table,
           Wq, bq, Wk, bk, Wv, bv):
    x_st = (jnp.zeros((2 * N_PAD, DH), jnp.float32)
            .at[:N].set(item_table[:, :DH])
            .at[N_PAD:N_PAD + N].set(item_table[:, DH:]))

    pad_e = E_PAD - E
    src = jnp.concatenate([edge_index[1], jnp.zeros((pad_e,), jnp.int32)])
    dst = jnp.concatenate([edge_index[0], jnp.zeros((pad_e,), jnp.int32)])
    w = jnp.concatenate([edge_weight, jnp.zeros((pad_e,), jnp.float32)])
    srcm = src.reshape(E_PAD // C, C)
    dstm = dst.reshape(E_PAD // C, C)
    wm = w

    items_p = jnp.zeros((B, 64), jnp.int32).at[:, :L].set(items).reshape(-1)
    alias_p = jnp.zeros((B, 64), jnp.int32).at[:, :L].set(alias_inputs)

    emb_st, _h1, seq_st = _graph_conv_sc(x_st, srcm, dstm, wm, items_p, alias_p)

    select = _attention_select(seq_st, Wq, bq, Wk, bk, Wv, bv)
    return _scores(select, emb_st)
